# baseline (speedup 1.0000x reference)
import functools

import jax
import jax.numpy as jnp
from jax.experimental import pallas as pl
from jax.experimental.pallas import tpu as pltpu

_N_SEG = 512
_N_FEAT = 256
_BLOCK_ROWS = 5000
_SEG_WIN = 128


def _body(bounds_ref, seg_ref, x_ref, w_ref, b_ref, out_ref, acc_ref):
    g = pl.program_id(0)
    smin = bounds_ref[0, g]
    smax = bounds_ref[1, g]
    x = x_ref[...]
    xb = x.astype(jnp.bfloat16)
    e = jax.lax.dot_general(
        xb, w_ref[...],
        (((1,), (0,)), ((), ())),
        preferred_element_type=jnp.float32,
    )
    e16 = (e + b_ref[...]).astype(jnp.bfloat16)
    a = jnp.exp(jnp.tanh(e16))
    v = jnp.concatenate([a, a * xb], axis=1)
    seg = seg_ref[0]

    @pl.when(g == 0)
    def _():
        acc_ref[...] = jnp.zeros_like(acc_ref)

    smax = jnp.minimum(smax, _N_SEG - 1)
    base = (smin // _SEG_WIN) * _SEG_WIN
    for k in range(_N_SEG // _SEG_WIN):

        @pl.when(smax >= base + k * _SEG_WIN)
        def _(k=k):
            lo = pl.multiple_of(base + k * _SEG_WIN, _SEG_WIN)
            sid = lo + jax.lax.broadcasted_iota(
                jnp.int32, (_SEG_WIN, seg.shape[1]), 0)
            onehot = (sid == seg).astype(jnp.bfloat16)
            part = jax.lax.dot_general(
                v, onehot, (((0,), (1,)), ((), ())),
                preferred_element_type=jnp.float32,
            )
            acc_ref[:, pl.ds(lo, _SEG_WIN)] += part

    @pl.when(g == pl.num_programs(0) - 1)
    def _():
        asum = acc_ref[:_N_FEAT, :_N_SEG]
        ax = acc_ref[_N_FEAT:, :_N_SEG]
        ratio = jnp.where(asum > 0.0, ax / asum, 0.0)
        out_ref[...] = jax.lax.transpose(ratio, (1, 0))


@jax.jit
def kernel(x, batch_segmentation, weight, bias):
    n, f = x.shape
    r = _BLOCK_ROWS
    pad = (-n) % r
    seg = batch_segmentation.astype(jnp.int32)
    if pad:
        x = jnp.pad(x, ((0, pad), (0, 0)))
        seg = jnp.pad(seg, (0, pad), constant_values=_N_SEG)
    g = (n + pad) // r
    seg2 = seg.reshape(g, r)
    bounds = jnp.stack([seg2[:, 0], seg2[:, -1]])
    seg3 = seg2.reshape(g, 1, r)
    w_bf = weight.astype(jnp.bfloat16)
    b2 = bias.reshape(1, f)

    grid_spec = pltpu.PrefetchScalarGridSpec(
        num_scalar_prefetch=1,
        grid=(g,),
        in_specs=[
            pl.BlockSpec((1, 1, r), lambda i, b: (i, 0, 0)),
            pl.BlockSpec((r, f), lambda i, b: (i, 0)),
            pl.BlockSpec((f, f), lambda i, b: (0, 0)),
            pl.BlockSpec((1, f), lambda i, b: (0, 0)),
        ],
        out_specs=pl.BlockSpec((_N_SEG, f), lambda i, b: (0, 0)),
        scratch_shapes=[
            pltpu.VMEM((2 * f, _N_SEG + _SEG_WIN), jnp.float32)],
    )
    return pl.pallas_call(
        _body,
        grid_spec=grid_spec,
        out_shape=jax.ShapeDtypeStruct((_N_SEG, f), jnp.float32),
        compiler_params=pltpu.CompilerParams(
            dimension_semantics=("arbitrary",),
        ),
    )(bounds, seg3, x, w_bf, b2)


# speedup vs baseline: 1.3268x; 1.3268x over previous
import functools

import jax
import jax.numpy as jnp
from jax.experimental import pallas as pl
from jax.experimental.pallas import tpu as pltpu

_N_SEG = 512
_N_FEAT = 256
_BLOCK_ROWS = 5000
_SEG_WIN = 128


def _body(bounds_ref, seg_ref, x_ref, w_ref, b_ref, out_ref, acc_ref):
    g = pl.program_id(0)
    smin = bounds_ref[0, g]
    smax = bounds_ref[1, g]
    x = x_ref[...]
    xb = x.astype(jnp.bfloat16)
    e = jax.lax.dot_general(
        xb, w_ref[...],
        (((1,), (0,)), ((), ())),
        preferred_element_type=jnp.float32,
    )
    e16 = (e + b_ref[...]).astype(jnp.bfloat16)
    a = jnp.exp(jnp.tanh(e16))
    v = jnp.concatenate([a, a * xb], axis=1)
    seg = seg_ref[0]

    @pl.when(g == 0)
    def _():
        acc_ref[...] = jnp.zeros_like(acc_ref)

    smax = jnp.minimum(smax, _N_SEG - 1)
    half = _SEG_WIN // 2
    base = (smin // half) * half
    for k in range(_N_SEG // _SEG_WIN):

        @pl.when(smax >= base + k * _SEG_WIN)
        def _(k=k):
            lo = pl.multiple_of(base + k * _SEG_WIN, half)
            sid = lo + jax.lax.broadcasted_iota(
                jnp.int32, (_SEG_WIN, seg.shape[1]), 0)
            onehot = (sid == seg).astype(jnp.bfloat16)
            part = jax.lax.dot_general(
                onehot, v, (((1,), (0,)), ((), ())),
                preferred_element_type=jnp.float32,
            )
            acc_ref[pl.ds(lo, _SEG_WIN), :] += part

    @pl.when(g == pl.num_programs(0) - 1)
    def _():
        asum = acc_ref[:_N_SEG, :_N_FEAT]
        ax = acc_ref[:_N_SEG, _N_FEAT:]
        out_ref[...] = jnp.where(asum > 0.0, ax / asum, 0.0)


@jax.jit
def kernel(x, batch_segmentation, weight, bias):
    n, f = x.shape
    r = _BLOCK_ROWS
    pad = (-n) % r
    seg = batch_segmentation.astype(jnp.int32)
    if pad:
        x = jnp.pad(x, ((0, pad), (0, 0)))
        seg = jnp.pad(seg, (0, pad), constant_values=_N_SEG)
    g = (n + pad) // r
    seg2 = seg.reshape(g, r)
    bounds = jnp.stack([seg2[:, 0], seg2[:, -1]])
    seg3 = seg2.reshape(g, 1, r)
    w_bf = weight.astype(jnp.bfloat16)
    b2 = bias.reshape(1, f)

    grid_spec = pltpu.PrefetchScalarGridSpec(
        num_scalar_prefetch=1,
        grid=(g,),
        in_specs=[
            pl.BlockSpec((1, 1, r), lambda i, b: (i, 0, 0)),
            pl.BlockSpec((r, f), lambda i, b: (i, 0)),
            pl.BlockSpec((f, f), lambda i, b: (0, 0)),
            pl.BlockSpec((1, f), lambda i, b: (0, 0)),
        ],
        out_specs=pl.BlockSpec((_N_SEG, f), lambda i, b: (0, 0)),
        scratch_shapes=[
            pltpu.VMEM((_N_SEG + _SEG_WIN, 2 * f), jnp.float32)],
    )
    return pl.pallas_call(
        _body,
        grid_spec=grid_spec,
        out_shape=jax.ShapeDtypeStruct((_N_SEG, f), jnp.float32),
        compiler_params=pltpu.CompilerParams(
            dimension_semantics=("arbitrary",),
            vmem_limit_bytes=120 * 1024 * 1024,
        ),
    )(bounds, seg3, x, w_bf, b2)
